# Initial kernel scaffold; baseline (speedup 1.0000x reference)
import functools

import jax
import jax.numpy as jnp
from jax import lax
from jax.experimental import pallas as pl
from jax.experimental.pallas import tpu as pltpu
from jax.experimental.pallas import tpu_sc as plsc

_NUM_CORES = 2
_NUM_SUBCORES = 16
_NW = _NUM_CORES * _NUM_SUBCORES

_B = 16384 * 50
_D = 32
_BPW = _B // _NW
_CH = 1024
_NCH = _BPW // _CH

_mesh = plsc.VectorSubcoreMesh(core_axis_name="c", subcore_axis_name="s")


@functools.partial(
    pl.kernel,
    out_type=jax.ShapeDtypeStruct((_B, _D), jnp.float32),
    mesh=_mesh,
    scratch_types=[
        pltpu.VMEM((_CH,), jnp.int32),
        pltpu.VMEM((_CH, _D), jnp.float32),
        pltpu.SemaphoreType.DMA,
    ],
)
def _gather_kernel(tok_hbm, table_hbm, out_hbm, idx_v, rows_v, sem):
    wid = lax.axis_index("s") * _NUM_CORES + lax.axis_index("c")
    base = wid * _BPW

    def body(i, carry):
        off = base + i * _CH
        pltpu.sync_copy(tok_hbm.at[pl.ds(off, _CH)], idx_v)
        pltpu.async_copy(table_hbm.at[idx_v], rows_v, sem).wait()
        pltpu.sync_copy(rows_v, out_hbm.at[pl.ds(off, _CH)])
        return carry

    lax.fori_loop(0, _NCH, body, 0)


def kernel(token_ids, weight):
    flat = token_ids.reshape(-1).astype(jnp.int32)
    out = _gather_kernel(flat, weight)
    return out.reshape(token_ids.shape + (weight.shape[1],))


# speedup vs baseline: 1.0946x; 1.0946x over previous
import functools

import jax
import jax.numpy as jnp
from jax import lax
from jax.experimental import pallas as pl
from jax.experimental.pallas import tpu as pltpu
from jax.experimental.pallas import tpu_sc as plsc

_NUM_CORES = 2
_NUM_SUBCORES = 16
_NW = _NUM_CORES * _NUM_SUBCORES

_B = 16384 * 50
_D = 32
_BPW = _B // _NW
_CH = 1024
_NCH = _BPW // _CH

_mesh = plsc.VectorSubcoreMesh(core_axis_name="c", subcore_axis_name="s")


@functools.partial(
    pl.kernel,
    out_type=jax.ShapeDtypeStruct((_B, _D), jnp.float32),
    mesh=_mesh,
    scratch_types=[
        pltpu.VMEM((_CH,), jnp.int32),
        pltpu.VMEM((_CH, _D), jnp.float32),
        pltpu.SemaphoreType.DMA,
    ],
    compiler_params=pltpu.CompilerParams(use_tc_tiling_on_sc=False),
)
def _gather_kernel(tok_hbm, table_hbm, out_hbm, idx_v, rows_v, sem):
    wid = lax.axis_index("s") * _NUM_CORES + lax.axis_index("c")
    base = wid * _BPW

    def body(i, carry):
        off = base + i * _CH
        pltpu.sync_copy(tok_hbm.at[pl.ds(off, _CH)], idx_v)
        pltpu.async_copy(table_hbm.at[idx_v], rows_v, sem).wait()
        pltpu.sync_copy(rows_v, out_hbm.at[pl.ds(off, _CH)])
        return carry

    lax.fori_loop(0, _NCH, body, 0)


def kernel(token_ids, weight):
    flat = token_ids.reshape(-1).astype(jnp.int32)
    out = _gather_kernel(flat, weight)
    return out.reshape(token_ids.shape + (weight.shape[1],))


# speedup vs baseline: 1.1135x; 1.0173x over previous
import functools

import jax
import jax.numpy as jnp
from jax import lax
from jax.experimental import pallas as pl
from jax.experimental.pallas import tpu as pltpu
from jax.experimental.pallas import tpu_sc as plsc

_NUM_CORES = 2
_NUM_SUBCORES = 16
_NW = _NUM_CORES * _NUM_SUBCORES

_B = 16384 * 50
_D = 32
_BPW = _B // _NW
_CH = 1280
_NCH = _BPW // _CH

_mesh = plsc.VectorSubcoreMesh(core_axis_name="c", subcore_axis_name="s")


@functools.partial(
    pl.kernel,
    out_type=jax.ShapeDtypeStruct((_B, _D), jnp.float32),
    mesh=_mesh,
    scratch_types=[
        pltpu.VMEM((_CH,), jnp.int32),
        pltpu.VMEM((_CH,), jnp.int32),
        pltpu.VMEM((_CH, _D), jnp.float32),
        pltpu.VMEM((_CH, _D), jnp.float32),
        pltpu.SemaphoreType.DMA,
        pltpu.SemaphoreType.DMA,
        pltpu.SemaphoreType.DMA,
        pltpu.SemaphoreType.DMA,
        pltpu.SemaphoreType.DMA,
        pltpu.SemaphoreType.DMA,
    ],
    compiler_params=pltpu.CompilerParams(use_tc_tiling_on_sc=False),
)
def _gather_kernel(tok_hbm, table_hbm, out_hbm, idx0, idx1, rows0, rows1,
                   si0, si1, sg0, sg1, ss0, ss1):
    wid = lax.axis_index("s") * _NUM_CORES + lax.axis_index("c")
    base = wid * _BPW

    idx = (idx0, idx1)
    rows = (rows0, rows1)
    si = (si0, si1)
    sg = (sg0, sg1)
    ss = (ss0, ss1)

    def idx_cp(g, b):
        return pltpu.make_async_copy(tok_hbm.at[wid, g], idx[b], si[b])

    def gather_cp(g, b):
        return pltpu.make_async_copy(table_hbm.at[idx[b]], rows[b], sg[b])

    def store_cp(g, b):
        return pltpu.make_async_copy(
            rows[b], out_hbm.at[pl.ds(base + g * _CH, _CH)], ss[b])

    idx_cp(0, 0).start()
    idx_cp(1, 1).start()
    idx_cp(0, 0).wait()
    gather_cp(0, 0).start()

    @pl.loop(0, _NCH, step=2)
    def step2(g0):
        for db in range(2):
            g = g0 + db
            b = db
            o = 1 - b

            @pl.when(jnp.logical_and(g >= 1, g + 1 < _NCH))
            def _():
                store_cp(g - 1, o).wait()

            @pl.when(g + 1 < _NCH)
            def _():
                idx_cp(g + 1, o).wait()
                gather_cp(g + 1, o).start()

            gather_cp(g, b).wait()

            @pl.when(g + 2 < _NCH)
            def _():
                idx_cp(g + 2, b).start()

            store_cp(g, b).start()

    store_cp(_NCH - 2, 0).wait()
    store_cp(_NCH - 1, 1).wait()


def kernel(token_ids, weight):
    flat = token_ids.reshape(_NW, _NCH, _CH).astype(jnp.int32)
    out = _gather_kernel(flat, weight)
    return out.reshape(token_ids.shape + (weight.shape[1],))


# speedup vs baseline: 1.6055x; 1.4418x over previous
import functools

import jax
import jax.numpy as jnp
from jax import lax
from jax.experimental import pallas as pl
from jax.experimental.pallas import tpu as pltpu
from jax.experimental.pallas import tpu_sc as plsc

_NUM_CORES = 2
_NUM_SUBCORES = 16
_NW = _NUM_CORES * _NUM_SUBCORES

_T = 16384
_S = 50
_D = 32
_CH = 1024
_NTG = _T // _CH
_NCHUNK = _S * _NTG
_JPW = _NCHUNK // _NW

_mesh = plsc.VectorSubcoreMesh(core_axis_name="c", subcore_axis_name="s")


@functools.partial(
    pl.kernel,
    out_type=jax.ShapeDtypeStruct((_S, 4, 128, 8, 128), jnp.float32),
    mesh=_mesh,
    scratch_types=[
        pltpu.VMEM((_CH,), jnp.int32),
        pltpu.VMEM((_CH,), jnp.int32),
        pltpu.VMEM((_CH, _D), jnp.float32),
        pltpu.VMEM((_CH, _D), jnp.float32),
        pltpu.VMEM((4, 8, 8, 128), jnp.float32),
        pltpu.SemaphoreType.DMA,
        pltpu.SemaphoreType.DMA,
        pltpu.SemaphoreType.DMA,
        pltpu.SemaphoreType.DMA,
        pltpu.SemaphoreType.DMA,
    ],
    compiler_params=pltpu.CompilerParams(
        use_tc_tiling_on_sc=False, needs_layout_passes=False),
)
def _gather_kernel(tok_hbm, table_hbm, out_hbm, idx0, idx1, rows0, rows1,
                   tr, si0, si1, sg0, sg1, st):
    wid = lax.axis_index("s") * _NUM_CORES + lax.axis_index("c")
    c0 = wid * _JPW

    idx = (idx0, idx1)
    rows = (rows0, rows1)
    si = (si0, si1)
    sg = (sg0, sg1)
    iota = lax.iota(jnp.int32, 16)

    def s_tg(j):
        c = c0 + j
        return c // _NTG, c % _NTG

    def idx_cp(j, b):
        s, tg = s_tg(j)
        return pltpu.make_async_copy(tok_hbm.at[s, tg], idx[b], si[b])

    def gather_cp(j, b):
        return pltpu.make_async_copy(table_hbm.at[idx[b]], rows[b], sg[b])

    def store_cp(j, dt):
        s, tg = s_tg(j)
        return pltpu.make_async_copy(
            tr.at[dt], out_hbm.at[s, dt, pl.ds(tg * 8, 8)], st)

    def transpose_chunk(b):
        def m_body(m, carry):
            dtile = m // 64
            tq = (m // 8) % 8
            d = m % 8
            col = lax.broadcast(dtile * 8 + d, (16,))
            row_base = tq * 128
            for t0 in range(0, 128, 16):
                r_idx = row_base + t0 + iota
                v = plsc.load_gather(rows[b], [r_idx, col])
                tr[dtile, tq, d, pl.ds(t0, 16)] = v
            return carry

        lax.fori_loop(0, 256, m_body, 0)

    idx_cp(0, 0).start()
    idx_cp(1, 1).start()
    idx_cp(0, 0).wait()
    gather_cp(0, 0).start()

    def step(j, b):
        o = 1 - b

        @pl.when(j + 1 < _JPW)
        def _():
            idx_cp(j + 1, o).wait()
            gather_cp(j + 1, o).start()

        gather_cp(j, b).wait()

        @pl.when(j + 2 < _JPW)
        def _():
            idx_cp(j + 2, b).start()

        @pl.when(j >= 1)
        def _():
            for dt in range(4):
                store_cp(j - 1, dt).wait()

        transpose_chunk(b)
        for dt in range(4):
            store_cp(j, dt).start()

    @pl.loop(0, _JPW - 1, step=2)
    def step2(j0):
        for db in range(2):
            step(j0 + db, db)

    step(_JPW - 1, (_JPW - 1) % 2)
    for dt in range(4):
        store_cp(_JPW - 1, dt).wait()


def kernel(token_ids, weight):
    tok3 = jnp.transpose(token_ids).reshape(_S, _NTG, _CH).astype(jnp.int32)
    out5 = _gather_kernel(tok3, weight)
    return out5.transpose(2, 4, 0, 1, 3).reshape(_T, _S, _D)


# speedup vs baseline: 2.3802x; 1.4825x over previous
import functools

import jax
import jax.numpy as jnp
from jax import lax
from jax.experimental import pallas as pl
from jax.experimental.pallas import tpu as pltpu
from jax.experimental.pallas import tpu_sc as plsc

_NUM_CORES = 2
_NUM_SUBCORES = 16
_NW = _NUM_CORES * _NUM_SUBCORES

_T = 16384
_S = 50
_D = 32
_CH = 1024
_NTG = _T // _CH
_NCHUNK = _S * _NTG
_JPW = _NCHUNK // _NW

_mesh = plsc.VectorSubcoreMesh(core_axis_name="c", subcore_axis_name="s")


@functools.partial(
    pl.kernel,
    out_type=jax.ShapeDtypeStruct((_S, 4, 131072), jnp.float32),
    mesh=_mesh,
    scratch_types=[
        pltpu.VMEM((_CH,), jnp.int32),
        pltpu.VMEM((_CH,), jnp.int32),
        pltpu.VMEM((_CH, _D), jnp.float32),
        pltpu.VMEM((_CH, _D), jnp.float32),
        pltpu.VMEM((4 * 8192,), jnp.float32),
        pltpu.SemaphoreType.DMA,
        pltpu.SemaphoreType.DMA,
        pltpu.SemaphoreType.DMA,
        pltpu.SemaphoreType.DMA,
        pltpu.SemaphoreType.DMA,
    ],
    compiler_params=pltpu.CompilerParams(
        use_tc_tiling_on_sc=False, needs_layout_passes=False),
)
def _gather_kernel(tok_hbm, table_hbm, out_hbm, idx0, idx1, rows0, rows1,
                   tr, si0, si1, sg0, sg1, st):
    wid = lax.axis_index("s") * _NUM_CORES + lax.axis_index("c")
    c0 = wid * _JPW

    idx = (idx0, idx1)
    rows = (rows0, rows1)
    si = (si0, si1)
    sg = (sg0, sg1)
    iota = lax.iota(jnp.int32, 16)

    cols = {}
    dsts = {}
    for c_half in (0, 16):
        for k in range(16):
            rot = (iota + k) % 16
            dg = c_half + rot
            cols[(c_half, k)] = dg
            dsts[(c_half, k)] = (dg // 8) * 8192 + (dg % 8) * 128 + iota

    def s_tg(j):
        c = c0 + j
        return c // _NTG, c % _NTG

    def idx_cp(j, b):
        s, tg = s_tg(j)
        return pltpu.make_async_copy(tok_hbm.at[s, tg], idx[b], si[b])

    def gather_cp(j, b):
        return pltpu.make_async_copy(table_hbm.at[idx[b]], rows[b], sg[b])

    def store_cp(j, dt):
        s, tg = s_tg(j)
        return pltpu.make_async_copy(
            tr.at[pl.ds(dt * 8192, 8192)],
            out_hbm.at[s, dt, pl.ds(tg * 8192, 8192)], st)

    def transpose_chunk(b):
        def t_body(t16, carry):
            t0 = t16 * 16
            tq = t0 // 128
            sbase = tq * 1024 + (t0 - tq * 128)
            r_idx = t0 + iota
            for c_half in (0, 16):
                for k in range(16):
                    v = plsc.load_gather(rows[b], [r_idx, cols[(c_half, k)]])
                    plsc.store_scatter(tr, [dsts[(c_half, k)] + sbase], v)
            return carry

        lax.fori_loop(0, 64, t_body, 0)

    idx_cp(0, 0).start()
    idx_cp(1, 1).start()
    idx_cp(0, 0).wait()
    gather_cp(0, 0).start()

    def step(j, b):
        o = 1 - b

        @pl.when(j + 1 < _JPW)
        def _():
            idx_cp(j + 1, o).wait()
            gather_cp(j + 1, o).start()

        gather_cp(j, b).wait()

        @pl.when(j + 2 < _JPW)
        def _():
            idx_cp(j + 2, b).start()

        @pl.when(j >= 1)
        def _():
            for dt in range(4):
                store_cp(j - 1, dt).wait()

        transpose_chunk(b)
        for dt in range(4):
            store_cp(j, dt).start()

    @pl.loop(0, _JPW - 1, step=2)
    def step2(j0):
        for db in range(2):
            step(j0 + db, db)

    step(_JPW - 1, (_JPW - 1) % 2)
    for dt in range(4):
        store_cp(_JPW - 1, dt).wait()


def kernel(token_ids, weight):
    tok3 = jnp.transpose(token_ids).reshape(_S, _NTG, _CH).astype(jnp.int32)
    out3 = _gather_kernel(tok3, weight)
    out5 = out3.reshape(_S, 4, 128, 8, 128)
    return out5.transpose(2, 4, 0, 1, 3).reshape(_T, _S, _D)


# speedup vs baseline: 2.8960x; 1.2167x over previous
import functools

import jax
import jax.numpy as jnp
from jax import lax
from jax.experimental import pallas as pl
from jax.experimental.pallas import tpu as pltpu
from jax.experimental.pallas import tpu_sc as plsc

_NUM_CORES = 2
_NUM_SUBCORES = 16
_NW = _NUM_CORES * _NUM_SUBCORES

_T = 16384
_S = 50
_D = 32
_CH = 1024
_NTG = _T // _CH
_NCHUNK = _S * _NTG
_JPW = _NCHUNK // _NW

_mesh = plsc.VectorSubcoreMesh(core_axis_name="c", subcore_axis_name="s")


@functools.partial(
    pl.kernel,
    out_type=jax.ShapeDtypeStruct((_S, 4, 131072), jnp.float32),
    mesh=_mesh,
    scratch_types=[
        pltpu.VMEM((_CH,), jnp.int32),
        pltpu.VMEM((_CH,), jnp.int32),
        pltpu.VMEM((_CH, _D), jnp.float32),
        pltpu.VMEM((_CH, _D), jnp.float32),
        pltpu.VMEM((4 * 8192,), jnp.float32),
        pltpu.SemaphoreType.DMA,
        pltpu.SemaphoreType.DMA,
        pltpu.SemaphoreType.DMA,
        pltpu.SemaphoreType.DMA,
        pltpu.SemaphoreType.DMA,
    ],
    compiler_params=pltpu.CompilerParams(
        use_tc_tiling_on_sc=False, needs_layout_passes=False),
)
def _gather_kernel(tok_hbm, table_hbm, out_hbm, idx0, idx1, rows0, rows1,
                   tr, si0, si1, sg0, sg1, st):
    wid = lax.axis_index("s") * _NUM_CORES + lax.axis_index("c")
    c0 = wid * _JPW

    idx = (idx0, idx1)
    rows = (rows0, rows1)
    si = (si0, si1)
    sg = (sg0, sg1)
    iota = lax.iota(jnp.int32, 16)

    cols = {}
    dsts = {}
    for c_half in (0, 16):
        for k in range(16):
            rot = (iota + k) % 16
            dg = c_half + rot
            cols[(c_half, k)] = dg
            dsts[(c_half, k)] = (dg // 8) * 8192 + (dg % 8) * 128 + iota

    def s_tg(j):
        c = c0 + j
        return c // _NTG, c % _NTG

    def idx_cp(j, b):
        s, tg = s_tg(j)
        return pltpu.make_async_copy(tok_hbm.at[s, tg], idx[b], si[b])

    def gather_cp(j, b):
        return pltpu.make_async_copy(table_hbm.at[idx[b]], rows[b], sg[b])

    def store_cp(j, dt):
        s, tg = s_tg(j)
        return pltpu.make_async_copy(
            tr.at[pl.ds(dt * 8192, 8192)],
            out_hbm.at[s, dt, pl.ds(tg * 8192, 8192)], st)

    def transpose_chunk(b):
        @plsc.parallel_loop(0, 64)
        def t_body(t16):
            t0 = t16 * 16
            tq = t0 // 128
            sbase = tq * 1024 + (t0 - tq * 128)
            r_idx = t0 + iota
            for c_half in (0, 16):
                for k in range(16):
                    v = plsc.load_gather(rows[b], [r_idx, cols[(c_half, k)]])
                    plsc.store_scatter(tr, [dsts[(c_half, k)] + sbase], v)

    idx_cp(0, 0).start()
    idx_cp(1, 1).start()
    idx_cp(0, 0).wait()
    gather_cp(0, 0).start()

    def step(j, b):
        o = 1 - b

        @pl.when(j + 1 < _JPW)
        def _():
            idx_cp(j + 1, o).wait()
            gather_cp(j + 1, o).start()

        gather_cp(j, b).wait()

        @pl.when(j + 2 < _JPW)
        def _():
            idx_cp(j + 2, b).start()

        @pl.when(j >= 1)
        def _():
            for dt in range(4):
                store_cp(j - 1, dt).wait()

        transpose_chunk(b)
        for dt in range(4):
            store_cp(j, dt).start()

    @pl.loop(0, _JPW - 1, step=2)
    def step2(j0):
        for db in range(2):
            step(j0 + db, db)

    step(_JPW - 1, (_JPW - 1) % 2)
    for dt in range(4):
        store_cp(_JPW - 1, dt).wait()


def kernel(token_ids, weight):
    tok3 = jnp.transpose(token_ids).reshape(_S, _NTG, _CH).astype(jnp.int32)
    out3 = _gather_kernel(tok3, weight)
    out5 = out3.reshape(_S, 4, 128, 8, 128)
    return out5.transpose(2, 4, 0, 1, 3).reshape(_T, _S, _D)
